# Initial kernel scaffold; baseline (speedup 1.0000x reference)
import jax
import jax.numpy as jnp
from jax.experimental import pallas as pl


def kernel(x, edge_index, edge_attr, params):
    raise NotImplementedError("write your pallas kernel here")


# speedup vs baseline: 5.9872x; 5.9872x over previous
import functools

import jax
import jax.numpy as jnp
from jax import lax
from jax.experimental import pallas as pl
from jax.experimental.pallas import tpu as pltpu
from jax.experimental.pallas import tpu_sc as plsc

_N = 10000
_E = 320000
_D = 128
_DE = 16
_DEA = 32
_NC = 2
_NS = 16
_NW = _NC * _NS
_K = 80
_EPT = _E // _NW
_NCH = _EPT // _K
_RPT = 624
_TAILR = _N - _NS * _RPT
_ZR = 16

_sc_mesh = plsc.VectorSubcoreMesh(core_axis_name="c", subcore_axis_name="s")


def _zero_shared(s, zbuf, sh_ref, width):
    zv = jnp.zeros((16,), jnp.float32)
    nlane = width // 16

    def _zfill(i, carry):
        zbuf[i // nlane, pl.ds((i % nlane) * 16, 16)] = zv
        return carry

    lax.fori_loop(0, _ZR * nlane, _zfill, 0)
    row0 = s * _RPT

    def _zcopy(k, carry):
        pltpu.sync_copy(zbuf, sh_ref.at[pl.ds(row0 + k * _ZR, _ZR)])
        return carry

    lax.fori_loop(0, _RPT // _ZR + jnp.where(s == _NS - 1, _TAILR // _ZR, 0),
                  _zcopy, 0)


def _write_out(c, s, sh_ref, out_hbm):
    row0 = s * _RPT
    pltpu.sync_copy(sh_ref.at[pl.ds(row0, _RPT)],
                    out_hbm.at[c, pl.ds(row0, _RPT)])

    @pl.when(s == _NS - 1)
    def _tail():
        pltpu.sync_copy(sh_ref.at[pl.ds(_NS * _RPT, _TAILR)],
                        out_hbm.at[c, pl.ds(_NS * _RPT, _TAILR)])


@functools.partial(
    pl.kernel,
    mesh=_sc_mesh,
    out_type=jax.ShapeDtypeStruct((_NC, _N, _D), jnp.float32),
    scratch_types=[
        pltpu.VMEM((_NCH, _K), jnp.int32),
        pltpu.VMEM((_NCH, _K), jnp.int32),
        pltpu.VMEM((_K, _D), jnp.float32),
        pltpu.VMEM((_ZR, _D), jnp.float32),
        pltpu.VMEM_SHARED((_N, _D), jnp.float32),
        pltpu.SemaphoreType.DMA,
    ],
)
def _spmm(h_hbm, src_hbm, dst_hbm, out_hbm, src_v, dst_v, rows_v, zbuf, g_sh, sem):
    c = lax.axis_index("c")
    s = lax.axis_index("s")
    wid = c * _NS + s
    _zero_shared(s, zbuf, g_sh, _D)
    plsc.subcore_barrier()

    pltpu.sync_copy(src_hbm.at[wid], src_v)
    pltpu.sync_copy(dst_hbm.at[wid], dst_v)

    def _step(j, carry):
        pltpu.async_copy(h_hbm.at[src_v.at[j]], rows_v, sem).wait()
        pltpu.sync_copy(rows_v, g_sh.at[dst_v.at[j]], add=True)
        return carry

    lax.fori_loop(0, _NCH, _step, 0)
    plsc.subcore_barrier()
    _write_out(c, s, g_sh, out_hbm)


@functools.partial(
    pl.kernel,
    mesh=_sc_mesh,
    out_type=jax.ShapeDtypeStruct((_NC, _N, _D), jnp.float32),
    scratch_types=[
        pltpu.VMEM((_NCH, _K), jnp.int32),
        pltpu.VMEM((_K, _D), jnp.float32),
        pltpu.VMEM((_ZR, _D), jnp.float32),
        pltpu.VMEM_SHARED((_N, _D), jnp.float32),
        pltpu.SemaphoreType.DMA,
    ],
)
def _epass(ea_hbm, dst_hbm, out_hbm, dst_v, rows_v, zbuf, e_sh, sem):
    c = lax.axis_index("c")
    s = lax.axis_index("s")
    wid = c * _NS + s
    _zero_shared(s, zbuf, e_sh, _D)
    plsc.subcore_barrier()

    pltpu.sync_copy(dst_hbm.at[wid], dst_v)
    ebase = wid * _EPT

    def _step(j, carry):
        pltpu.sync_copy(ea_hbm.at[pl.ds(ebase + j * _K, _K)], rows_v)
        pltpu.sync_copy(rows_v, e_sh.at[dst_v.at[j]], add=True)
        return carry

    lax.fori_loop(0, _NCH, _step, 0)
    plsc.subcore_barrier()
    _write_out(c, s, e_sh, out_hbm)


_BLK = 1000
_NBLK = _N // _BLK


def _tc_layer(G, Eaug, Wn, We, bnbe, wb, priors, emit_combo):
    nprior = len(priors)

    def body(*refs):
        g_ref, e_ref, wn_ref, we_ref, bb_ref, wb_ref = refs[:6]
        prefs = refs[6:6 + nprior]
        orefs = refs[6 + nprior:]
        g = g_ref[0] + g_ref[1]
        e = e_ref[0] + e_ref[1]
        bias = jnp.dot(e[:, :_DE], we_ref[...], preferred_element_type=jnp.float32)
        bias = bias + e[:, _DE:_DE + 1] * bb_ref[...]
        x = jnp.dot(g, wn_ref[...], preferred_element_type=jnp.float32) + bias
        x = jnp.maximum(x, 0.0)
        orefs[0][...] = x
        if emit_combo:
            acc = x * wb_ref[0:1, :]
            for j in range(nprior):
                acc = acc + prefs[j][...] * wb_ref[j + 1:j + 2, :]
            orefs[1][...] = acc

    in_specs = [
        pl.BlockSpec((_NC, _BLK, _D), lambda i: (0, i, 0)),
        pl.BlockSpec((_NC, _BLK, _D), lambda i: (0, i, 0)),
        pl.BlockSpec((_D, _D), lambda i: (0, 0)),
        pl.BlockSpec((_DE, _D), lambda i: (0, 0)),
        pl.BlockSpec((1, _D), lambda i: (0, 0)),
        pl.BlockSpec((8, _D), lambda i: (0, 0)),
    ] + [pl.BlockSpec((_BLK, _D), lambda i: (i, 0)) for _ in range(nprior)]
    nout = 2 if emit_combo else 1
    out_shape = [jax.ShapeDtypeStruct((_N, _D), jnp.float32)] * nout
    out_specs = [pl.BlockSpec((_BLK, _D), lambda i: (i, 0)) for _ in range(nout)]
    return pl.pallas_call(
        body,
        grid=(_NBLK,),
        in_specs=in_specs,
        out_specs=out_specs,
        out_shape=out_shape,
    )(G, Eaug, Wn, We, bnbe, wb, *priors)


def kernel(x, edge_index, edge_attr, params):
    src = edge_index[0].reshape(_NW, _NCH, _K)
    dst = edge_index[1].reshape(_NW, _NCH, _K)
    ea = jnp.concatenate(
        [
            edge_attr,
            jnp.ones((_E, 1), jnp.float32),
            jnp.zeros((_E, _D - _DE - 1), jnp.float32),
        ],
        axis=1,
    )
    Eaug = _epass(ea, dst)
    L = params["layers"]
    w = params["skip"]
    ones_row = jnp.ones((1, _D), jnp.float32)

    def lay(i, h, wvals, priors):
        p = L[i]
        G = _spmm(h, src, dst)
        bnbe = (p["bn"] + p["be"]).reshape(1, _D)
        emit = wvals is not None
        if emit:
            pad = [jnp.float32(0.0)] * (8 - len(wvals))
            wb = jnp.stack(list(wvals) + pad)[:, None] * ones_row
        else:
            wb = jnp.zeros((8, _D), jnp.float32)
        return _tc_layer(G, Eaug, p["Wn"], p["We"], bnbe, wb, priors, emit)

    (x1,) = lay(0, x, None, [])
    x2, h3 = lay(1, x1, [w["w2_2"], w["w2_1"]], [x1])
    x3, h4 = lay(2, h3, [w["w3_3"], w["w3_1"], w["w3_2"]], [x1, h3])
    x4, h5 = lay(3, h4, [w["w4_4"], w["w4_1"], w["w4_2"], w["w4_3"]], [x1, h3, h4])
    x5, h6 = lay(3, h5, [w["w5_5"], w["w5_1"], w["w5_2"], w["w5_3"], w["w5_4"]],
                 [x1, h3, h4, h5])
    x6, h7 = lay(4, h6, [w["w6_6"], w["w6_1"], w["w6_2"], w["w6_3"], w["w6_4"],
                         w["w6_5"]], [x1, h3, h4, h5, h6])
    x7, h8 = lay(5, h7, [w["w7_7"], w["w7_1"], w["w7_2"], w["w7_3"], w["w7_4"],
                         w["w7_5"], w["w7_6"]], [x1, h3, h4, h5, h6, h7])
    (out,) = lay(7, h8, None, [])
    return out


# speedup vs baseline: 9.4363x; 1.5761x over previous
import functools

import jax
import jax.numpy as jnp
from jax import lax
from jax.experimental import pallas as pl
from jax.experimental.pallas import tpu as pltpu
from jax.experimental.pallas import tpu_sc as plsc

_N = 10000
_E = 320000
_D = 128
_DE = 16
_DEA = 32
_NC = 2
_NS = 16
_NW = _NC * _NS
_K = 80
_EPT = _E // _NW
_NCH = _EPT // _K
_RPT = 624
_TAILR = _N - _NS * _RPT
_ZR = 8

_sc_mesh = plsc.VectorSubcoreMesh(core_axis_name="c", subcore_axis_name="s")


def _zero_shared(s, zbuf, sh_ref, width):
    zv = jnp.zeros((16,), jnp.float32)
    nlane = width // 16

    def _zfill(i, carry):
        zbuf[i // nlane, pl.ds((i % nlane) * 16, 16)] = zv
        return carry

    lax.fori_loop(0, _ZR * nlane, _zfill, 0)
    row0 = s * _RPT

    def _zcopy(k, carry):
        pltpu.sync_copy(zbuf, sh_ref.at[pl.ds(row0 + k * _ZR, _ZR)])
        return carry

    lax.fori_loop(0, _RPT // _ZR + jnp.where(s == _NS - 1, _TAILR // _ZR, 0),
                  _zcopy, 0)


def _write_out(c, s, sh_ref, out_hbm):
    row0 = s * _RPT
    pltpu.sync_copy(sh_ref.at[pl.ds(row0, _RPT)],
                    out_hbm.at[c, pl.ds(row0, _RPT)])

    @pl.when(s == _NS - 1)
    def _tail():
        pltpu.sync_copy(sh_ref.at[pl.ds(_NS * _RPT, _TAILR)],
                        out_hbm.at[c, pl.ds(_NS * _RPT, _TAILR)])


@functools.partial(
    pl.kernel,
    mesh=_sc_mesh,
    out_type=jax.ShapeDtypeStruct((_NC, _N, _D), jnp.float32),
    scratch_types=[
        pltpu.VMEM((_NCH, _K), jnp.int32),
        pltpu.VMEM((2, _K), jnp.int32),
        pltpu.VMEM((_K, _D), jnp.float32),
        pltpu.VMEM((_K, _D), jnp.float32),
        pltpu.VMEM((_ZR, _D), jnp.float32),
        pltpu.VMEM_SHARED((_N, _D), jnp.float32),
        pltpu.SemaphoreType.DMA,
        pltpu.SemaphoreType.DMA,
        pltpu.SemaphoreType.DMA,
        pltpu.SemaphoreType.DMA,
    ],
)
def _spmm(h_hbm, src_hbm, dst_hbm, out_hbm, src_v, dst_r, rows0, rows1, zbuf,
          g_sh, semg0, semg1, semd0, semd1):
    c = lax.axis_index("c")
    s = lax.axis_index("s")
    wid = c * _NS + s
    _zero_shared(s, zbuf, g_sh, _D)
    plsc.subcore_barrier()

    pltpu.sync_copy(src_hbm.at[wid], src_v)

    def _start(j, b, buf, semg, semd):
        pltpu.async_copy(dst_hbm.at[wid, j], dst_r.at[pl.ds(b, 1)], semd)
        pltpu.async_copy(h_hbm.at[src_v.at[j]], buf, semg)

    def _finish(j, b, buf, semg, semd):
        pltpu.make_async_copy(dst_hbm.at[wid, j], dst_r.at[pl.ds(b, 1)],
                              semd).wait()
        pltpu.make_async_copy(h_hbm.at[src_v.at[j]], buf, semg).wait()
        pltpu.sync_copy(buf, g_sh.at[dst_r.at[b]], add=True)

    _start(0, 0, rows0, semg0, semd0)
    _start(1, 1, rows1, semg1, semd1)

    def _pair(i, carry):
        j = 2 * i
        _finish(j, 0, rows0, semg0, semd0)
        _start(j + 2, 0, rows0, semg0, semd0)
        _finish(j + 1, 1, rows1, semg1, semd1)
        _start(j + 3, 1, rows1, semg1, semd1)
        return carry

    lax.fori_loop(0, (_NCH - 3) // 2, _pair, 0)
    _finish(_NCH - 3, 0, rows0, semg0, semd0)
    _finish(_NCH - 2, 1, rows1, semg1, semd1)
    _start(_NCH - 1, 0, rows0, semg0, semd0)
    _finish(_NCH - 1, 0, rows0, semg0, semd0)
    plsc.subcore_barrier()
    _write_out(c, s, g_sh, out_hbm)


@functools.partial(
    pl.kernel,
    mesh=_sc_mesh,
    out_type=jax.ShapeDtypeStruct((_NC, _N, _D), jnp.float32),
    scratch_types=[
        pltpu.VMEM((2, _K), jnp.int32),
        pltpu.VMEM((_K, _D), jnp.float32),
        pltpu.VMEM((_K, _D), jnp.float32),
        pltpu.VMEM((_ZR, _D), jnp.float32),
        pltpu.VMEM_SHARED((_N, _D), jnp.float32),
        pltpu.SemaphoreType.DMA,
        pltpu.SemaphoreType.DMA,
        pltpu.SemaphoreType.DMA,
        pltpu.SemaphoreType.DMA,
    ],
)
def _epass(ea_hbm, dst_hbm, out_hbm, dst_r, rows0, rows1, zbuf, e_sh,
           semg0, semg1, semd0, semd1):
    c = lax.axis_index("c")
    s = lax.axis_index("s")
    wid = c * _NS + s
    _zero_shared(s, zbuf, e_sh, _D)
    plsc.subcore_barrier()

    ebase = wid * _EPT

    def _start(j, b, buf, semg, semd):
        pltpu.async_copy(dst_hbm.at[wid, j], dst_r.at[pl.ds(b, 1)], semd)
        pltpu.async_copy(ea_hbm.at[pl.ds(ebase + j * _K, _K)], buf, semg)

    def _finish(j, b, buf, semg, semd):
        pltpu.make_async_copy(dst_hbm.at[wid, j], dst_r.at[pl.ds(b, 1)],
                              semd).wait()
        pltpu.make_async_copy(ea_hbm.at[pl.ds(ebase + j * _K, _K)], buf,
                              semg).wait()
        pltpu.sync_copy(buf, e_sh.at[dst_r.at[b]], add=True)

    _start(0, 0, rows0, semg0, semd0)
    _start(1, 1, rows1, semg1, semd1)

    def _pair(i, carry):
        j = 2 * i
        _finish(j, 0, rows0, semg0, semd0)
        _start(j + 2, 0, rows0, semg0, semd0)
        _finish(j + 1, 1, rows1, semg1, semd1)
        _start(j + 3, 1, rows1, semg1, semd1)
        return carry

    lax.fori_loop(0, (_NCH - 3) // 2, _pair, 0)
    _finish(_NCH - 3, 0, rows0, semg0, semd0)
    _finish(_NCH - 2, 1, rows1, semg1, semd1)
    _start(_NCH - 1, 0, rows0, semg0, semd0)
    _finish(_NCH - 1, 0, rows0, semg0, semd0)
    plsc.subcore_barrier()
    _write_out(c, s, e_sh, out_hbm)


_BLK = 1000
_NBLK = _N // _BLK


def _tc_layer(G, Eaug, Wn, We, bnbe, wb, priors, emit_combo):
    nprior = len(priors)

    def body(*refs):
        g_ref, e_ref, wn_ref, we_ref, bb_ref, wb_ref = refs[:6]
        prefs = refs[6:6 + nprior]
        orefs = refs[6 + nprior:]
        g = g_ref[0] + g_ref[1]
        e = e_ref[0] + e_ref[1]
        bias = jnp.dot(e[:, :_DE], we_ref[...], preferred_element_type=jnp.float32)
        bias = bias + e[:, _DE:_DE + 1] * bb_ref[...]
        x = jnp.dot(g, wn_ref[...], preferred_element_type=jnp.float32) + bias
        x = jnp.maximum(x, 0.0)
        orefs[0][...] = x
        if emit_combo:
            acc = x * wb_ref[0:1, :]
            for j in range(nprior):
                acc = acc + prefs[j][...] * wb_ref[j + 1:j + 2, :]
            orefs[1][...] = acc

    in_specs = [
        pl.BlockSpec((_NC, _BLK, _D), lambda i: (0, i, 0)),
        pl.BlockSpec((_NC, _BLK, _D), lambda i: (0, i, 0)),
        pl.BlockSpec((_D, _D), lambda i: (0, 0)),
        pl.BlockSpec((_DE, _D), lambda i: (0, 0)),
        pl.BlockSpec((1, _D), lambda i: (0, 0)),
        pl.BlockSpec((8, _D), lambda i: (0, 0)),
    ] + [pl.BlockSpec((_BLK, _D), lambda i: (i, 0)) for _ in range(nprior)]
    nout = 2 if emit_combo else 1
    out_shape = [jax.ShapeDtypeStruct((_N, _D), jnp.float32)] * nout
    out_specs = [pl.BlockSpec((_BLK, _D), lambda i: (i, 0)) for _ in range(nout)]
    return pl.pallas_call(
        body,
        grid=(_NBLK,),
        in_specs=in_specs,
        out_specs=out_specs,
        out_shape=out_shape,
    )(G, Eaug, Wn, We, bnbe, wb, *priors)


def kernel(x, edge_index, edge_attr, params):
    src = edge_index[0].reshape(_NW, _NCH, _K)
    dst = edge_index[1].reshape(_NW, _NCH, 1, _K)
    ea = jnp.concatenate(
        [
            edge_attr,
            jnp.ones((_E, 1), jnp.float32),
            jnp.zeros((_E, _D - _DE - 1), jnp.float32),
        ],
        axis=1,
    )
    Eaug = _epass(ea, dst)
    L = params["layers"]
    w = params["skip"]
    ones_row = jnp.ones((1, _D), jnp.float32)

    def lay(i, h, wvals, priors):
        p = L[i]
        G = _spmm(h, src, dst)
        bnbe = (p["bn"] + p["be"]).reshape(1, _D)
        emit = wvals is not None
        if emit:
            pad = [jnp.float32(0.0)] * (8 - len(wvals))
            wb = jnp.stack(list(wvals) + pad)[:, None] * ones_row
        else:
            wb = jnp.zeros((8, _D), jnp.float32)
        return _tc_layer(G, Eaug, p["Wn"], p["We"], bnbe, wb, priors, emit)

    (x1,) = lay(0, x, None, [])
    x2, h3 = lay(1, x1, [w["w2_2"], w["w2_1"]], [x1])
    x3, h4 = lay(2, h3, [w["w3_3"], w["w3_1"], w["w3_2"]], [x1, h3])
    x4, h5 = lay(3, h4, [w["w4_4"], w["w4_1"], w["w4_2"], w["w4_3"]], [x1, h3, h4])
    x5, h6 = lay(3, h5, [w["w5_5"], w["w5_1"], w["w5_2"], w["w5_3"], w["w5_4"]],
                 [x1, h3, h4, h5])
    x6, h7 = lay(4, h6, [w["w6_6"], w["w6_1"], w["w6_2"], w["w6_3"], w["w6_4"],
                         w["w6_5"]], [x1, h3, h4, h5, h6])
    x7, h8 = lay(5, h7, [w["w7_7"], w["w7_1"], w["w7_2"], w["w7_3"], w["w7_4"],
                         w["w7_5"], w["w7_6"]], [x1, h3, h4, h5, h6, h7])
    (out,) = lay(7, h8, None, [])
    return out
